# baseline (speedup 1.0000x reference)
import functools

import jax
import jax.numpy as jnp
from jax import lax
from jax.experimental import pallas as pl
from jax.experimental.pallas import tpu as pltpu
from jax.experimental.pallas import tpu_sc as plsc

N_NODES = 100000
N_EDGES = 3200000
N_GRAPHS = 512
F_IN = 16

NS = 16
EDGES_PER_CALL = N_EDGES // 2

EDGES_PER_TILE = EDGES_PER_CALL // NS
ECHUNK = 800
N_ECHUNKS = EDGES_PER_TILE // ECHUNK

RCHUNK = 800
N_RCHUNKS = N_NODES // RCHUNK
MAX_RCHUNKS_PER_TILE = (N_RCHUNKS + NS - 1) // NS

BLK = 1024
NB = (N_NODES + BLK - 1) // BLK
NPAD = NB * BLK
FW = 48


def _sc_body(src_hbm, dst_hbm, x_hbm, agg_out, cnt_out,
             agg_sh, cnt_sh, src_v, dst_v, rows_v, ones_v,
             cnt_stage, sem):
    s = lax.axis_index("s")

    def _zero_rows(r, carry):
        rows_v[r, :] = jnp.zeros((16,), jnp.float32)
        return carry
    lax.fori_loop(0, RCHUNK, _zero_rows, 0)

    def _fill_1d(r, carry):
        ones_v[pl.ds(r * 16, 16)] = jnp.ones((16,), jnp.float32)
        cnt_stage[pl.ds(r * 16, 16)] = jnp.zeros((16,), jnp.float32)
        return carry
    lax.fori_loop(0, ECHUNK // 16, _fill_1d, 0)

    for k in range(MAX_RCHUNKS_PER_TILE):
        m = s + NS * k

        @pl.when(m < N_RCHUNKS)
        def _():
            r0 = pl.multiple_of(m * RCHUNK, 8)
            pltpu.sync_copy(rows_v, agg_sh.at[pl.ds(r0, RCHUNK)])
            pltpu.sync_copy(cnt_stage, cnt_sh.at[pl.ds(r0, RCHUNK)])

    plsc.subcore_barrier()

    def _edge_chunk(j, carry):
        base = pl.multiple_of(s * EDGES_PER_TILE + j * ECHUNK, 8)
        pltpu.sync_copy(src_hbm.at[pl.ds(base, ECHUNK)], src_v)
        pltpu.sync_copy(dst_hbm.at[pl.ds(base, ECHUNK)], dst_v)
        pltpu.async_copy(x_hbm.at[src_v], rows_v, sem).wait()
        pltpu.sync_copy(rows_v, agg_sh.at[dst_v], add=True)
        pltpu.sync_copy(ones_v, cnt_sh.at[dst_v], add=True)
        return carry
    lax.fori_loop(0, N_ECHUNKS, _edge_chunk, 0)

    plsc.subcore_barrier()

    for k in range(MAX_RCHUNKS_PER_TILE):
        m = s + NS * k

        @pl.when(m < N_RCHUNKS)
        def _():
            r0 = pl.multiple_of(m * RCHUNK, 8)
            pltpu.sync_copy(agg_sh.at[pl.ds(r0, RCHUNK)], rows_v)
            pltpu.sync_copy(rows_v, agg_out.at[pl.ds(r0, RCHUNK)])
            pltpu.sync_copy(cnt_sh.at[pl.ds(r0, RCHUNK)], cnt_stage)
            pltpu.sync_copy(cnt_stage, cnt_out.at[pl.ds(r0, RCHUNK)])


_sc_aggregate = functools.partial(
    pl.kernel,
    out_type=[
        jax.ShapeDtypeStruct((N_NODES, F_IN), jnp.float32),
        jax.ShapeDtypeStruct((N_NODES,), jnp.float32),
    ],
    mesh=plsc.VectorSubcoreMesh(core_axis_name="c", subcore_axis_name="s",
                                num_cores=1),
    scratch_types=[
        pltpu.VMEM_SHARED((N_NODES, F_IN), jnp.float32),
        pltpu.VMEM_SHARED((N_NODES,), jnp.float32),
        pltpu.VMEM((ECHUNK,), jnp.int32),
        pltpu.VMEM((ECHUNK,), jnp.int32),
        pltpu.VMEM((ECHUNK, F_IN), jnp.float32),
        pltpu.VMEM((ECHUNK,), jnp.float32),
        pltpu.VMEM((RCHUNK,), jnp.float32),
        pltpu.SemaphoreType.DMA,
    ],
    compiler_params=pltpu.CompilerParams(use_tc_tiling_on_sc=False),
)(_sc_body)


def _tc_body(agg_a_ref, agg_b_ref, cnt_a_ref, cnt_b_ref, x_ref, batch_ref,
             wl_ref, bl_ref, wr_ref, wlin_ref, blin_ref, out_ref, acc_ref):
    i = pl.program_id(0)

    @pl.when(i == 0)
    def _():
        acc_ref[...] = jnp.zeros_like(acc_ref)

    agg = agg_a_ref[...] + agg_b_ref[...]
    cnt = cnt_a_ref[...] + cnt_b_ref[...]
    mean = agg / jnp.maximum(cnt, 1.0)
    ids = batch_ref[0]

    node = i * BLK + lax.broadcasted_iota(jnp.int32, (BLK, 1), 0)
    valid = node < N_NODES

    gid = lax.broadcasted_iota(jnp.int32, (BLK, N_GRAPHS), 1)
    onehot = jnp.where((ids == gid) & valid, 1.0, 0.0)

    feat = jnp.concatenate(
        [mean, x_ref[...], jnp.ones((BLK, 1), jnp.float32),
         jnp.zeros((BLK, FW - 2 * F_IN - 1), jnp.float32)], axis=1)
    feat = jnp.where(valid, feat, 0.0)

    acc_ref[...] += lax.dot_general(
        onehot, feat, (((0,), (0,)), ((), ())),
        preferred_element_type=jnp.float32)

    @pl.when(i == NB - 1)
    def _():
        acc = acc_ref[...]
        pmean = acc[:, 0:F_IN]
        px = acc[:, F_IN:2 * F_IN]
        gcnt = acc[:, 2 * F_IN:2 * F_IN + 1]
        gsum = (
            lax.dot_general(pmean, wl_ref[...], (((1,), (1,)), ((), ())),
                            preferred_element_type=jnp.float32)
            + lax.dot_general(px, wr_ref[...], (((1,), (1,)), ((), ())),
                              preferred_element_type=jnp.float32)
            + gcnt * bl_ref[...])
        gmean = gsum / jnp.maximum(gcnt, 1.0)
        out_ref[...] = (
            lax.dot_general(gmean, wlin_ref[...], (((1,), (1,)), ((), ())),
                            preferred_element_type=jnp.float32)
            + blin_ref[...])


def _tc_finish(agg_a, agg_b, cnt_a, cnt_b, x, batch3,
               w_l, b_l, w_r, w_lin, b_lin):
    return pl.pallas_call(
        _tc_body,
        grid=(NB,),
        in_specs=[
            pl.BlockSpec((BLK, F_IN), lambda i: (i, 0)),
            pl.BlockSpec((BLK, F_IN), lambda i: (i, 0)),
            pl.BlockSpec((BLK, 1), lambda i: (i, 0)),
            pl.BlockSpec((BLK, 1), lambda i: (i, 0)),
            pl.BlockSpec((BLK, F_IN), lambda i: (i, 0)),
            pl.BlockSpec((1, BLK, 1), lambda i: (i, 0, 0)),
            pl.BlockSpec((32, 16), lambda i: (0, 0)),
            pl.BlockSpec((1, 32), lambda i: (0, 0)),
            pl.BlockSpec((32, 16), lambda i: (0, 0)),
            pl.BlockSpec((64, 32), lambda i: (0, 0)),
            pl.BlockSpec((1, 64), lambda i: (0, 0)),
        ],
        out_specs=pl.BlockSpec((N_GRAPHS, 64), lambda i: (0, 0)),
        out_shape=jax.ShapeDtypeStruct((N_GRAPHS, 64), jnp.float32),
        scratch_shapes=[pltpu.VMEM((N_GRAPHS, FW), jnp.float32)],
    )(agg_a, agg_b, cnt_a, cnt_b, x, batch3, w_l, b_l, w_r, w_lin, b_lin)


def kernel(x, edge_index, batch, W_l, b_l, W_r, W_lin, b_lin):
    src = edge_index[0].astype(jnp.int32)
    dst = edge_index[1].astype(jnp.int32)
    agg_a, cnt_a = _sc_aggregate(src[:EDGES_PER_CALL], dst[:EDGES_PER_CALL], x)
    agg_b, cnt_b = _sc_aggregate(src[EDGES_PER_CALL:], dst[EDGES_PER_CALL:], x)
    batch_pad = jnp.concatenate(
        [batch.astype(jnp.int32),
         jnp.full((NPAD - N_NODES,), N_GRAPHS, jnp.int32)]).reshape(NB, BLK, 1)
    return _tc_finish(agg_a, agg_b, cnt_a.reshape(N_NODES, 1),
                      cnt_b.reshape(N_NODES, 1), x, batch_pad,
                      W_l, b_l.reshape(1, 32), W_r, W_lin, b_lin.reshape(1, 64))


# speedup vs baseline: 1.8767x; 1.8767x over previous
import functools

import jax
import jax.numpy as jnp
from jax import lax
from jax.experimental import pallas as pl
from jax.experimental.pallas import tpu as pltpu
from jax.experimental.pallas import tpu_sc as plsc

N_NODES = 100000
N_EDGES = 3200000
N_GRAPHS = 512
F_IN = 16

NC = 2
NS = 16
NW = NC * NS

ECHUNK = 512
N_CHUNKS = N_EDGES // ECHUNK
MAX_K = (N_CHUNKS + NW - 1) // NW
T_OUTER = MAX_K // 4

RCHUNK = 400
N_RCHUNKS = N_NODES // RCHUNK
MAX_RCHUNKS_PER_TILE = (N_RCHUNKS + NS - 1) // NS

BLK = 1024
NB = (N_NODES + BLK - 1) // BLK
NPAD = NB * BLK
FW = 48


def _sc_body(src_hbm, dst_hbm, x_hbm, agg_out, cnt_out,
             agg_sh, cnt_sh,
             src_v0, src_v1, src_v2, src_v3,
             dst_v0, dst_v1, dst_v2, dst_v3,
             rows_v0, rows_v1, ones_v, cnt_stage,
             semi0, semi1, semi2, semi3, semg0, semg1, sems0, sems1):
    c = lax.axis_index("c")
    s = lax.axis_index("s")
    wid = s * NC + c

    SRC = [src_v0, src_v1, src_v2, src_v3]
    DST = [dst_v0, dst_v1, dst_v2, dst_v3]
    ROWS = [rows_v0, rows_v1]
    SEMI = [semi0, semi1, semi2, semi3]
    SEMG = [semg0, semg1]
    SEMS = [sems0, sems1]

    nk = (N_CHUNKS - 1 - wid) // NW + 1

    def _zero_rows(r, carry):
        rows_v0[r, :] = jnp.zeros((16,), jnp.float32)
        return carry
    lax.fori_loop(0, RCHUNK, _zero_rows, 0)

    def _fill_1d(r, carry):
        ones_v[pl.ds(r * 16, 16)] = jnp.ones((16,), jnp.float32)
        return carry
    lax.fori_loop(0, ECHUNK // 16, _fill_1d, 0)

    def _zero_cs(r, carry):
        cnt_stage[pl.ds(r * 16, 16)] = jnp.zeros((16,), jnp.float32)
        return carry
    lax.fori_loop(0, RCHUNK // 16, _zero_cs, 0)

    zrows = rows_v0.at[pl.ds(0, RCHUNK)]

    for k in range(MAX_RCHUNKS_PER_TILE):
        m = s + NS * k

        @pl.when(m < N_RCHUNKS)
        def _():
            r0 = pl.multiple_of(m * RCHUNK, 8)
            pltpu.sync_copy(zrows, agg_sh.at[pl.ds(r0, RCHUNK)])
            pltpu.sync_copy(cnt_stage, cnt_sh.at[pl.ds(r0, RCHUNK)])

    plsc.subcore_barrier()

    def chunk_base(k):
        return pl.multiple_of((wid + NW * k) * ECHUNK, 8)

    def issue_idx(k, ib):
        b0 = chunk_base(k)
        pltpu.async_copy(src_hbm.at[pl.ds(b0, ECHUNK)], SRC[ib], SEMI[ib])
        pltpu.async_copy(dst_hbm.at[pl.ds(b0, ECHUNK)], DST[ib], SEMI[ib])

    def wait_idx(ib):
        pltpu.make_async_copy(src_hbm.at[pl.ds(0, ECHUNK)], SRC[ib],
                              SEMI[ib]).wait()
        pltpu.make_async_copy(dst_hbm.at[pl.ds(0, ECHUNK)], DST[ib],
                              SEMI[ib]).wait()

    def issue_gather(ib, rb):
        pltpu.async_copy(x_hbm.at[SRC[ib]], ROWS[rb], SEMG[rb])

    def wait_gather(ib, rb):
        pltpu.make_async_copy(x_hbm.at[SRC[ib]], ROWS[rb], SEMG[rb]).wait()

    def issue_scatter(ib, rb):
        pltpu.async_copy(ROWS[rb], agg_sh.at[DST[ib]], SEMS[rb], add=True)
        pltpu.async_copy(ones_v, cnt_sh.at[DST[ib]], SEMS[rb], add=True)

    def wait_scatter(ib, rb):
        pltpu.make_async_copy(ROWS[rb], agg_sh.at[DST[ib]], SEMS[rb]).wait()
        pltpu.make_async_copy(ones_v, cnt_sh.at[DST[ib]], SEMS[rb]).wait()

    issue_idx(0, 0)
    issue_idx(1, 1)
    issue_idx(2, 2)
    wait_idx(0)
    issue_gather(0, 0)

    def _pipe_step(t, carry):
        for b in range(4):
            k = 4 * t + b

            @pl.when((k >= 1) & (k - 1 < nk))
            def _():
                wait_scatter((b - 1) % 4, (b - 1) % 2)

            @pl.when(k + 3 < nk)
            def _():
                issue_idx(k + 3, (b + 3) % 4)

            @pl.when(k + 1 < nk)
            def _():
                wait_idx((b + 1) % 4)
                issue_gather((b + 1) % 4, (b + 1) % 2)

            @pl.when(k < nk)
            def _():
                wait_gather(b % 4, b % 2)
                issue_scatter(b % 4, b % 2)
        return carry
    lax.fori_loop(0, T_OUTER, _pipe_step, 0)

    @pl.when(nk > MAX_K - 1)
    def _():
        wait_scatter(3, 1)

    plsc.subcore_barrier()

    for k in range(MAX_RCHUNKS_PER_TILE):
        m = s + NS * k

        @pl.when(m < N_RCHUNKS)
        def _():
            r0 = pl.multiple_of(m * RCHUNK, 8)
            pltpu.sync_copy(agg_sh.at[pl.ds(r0, RCHUNK)], zrows)
            pltpu.sync_copy(zrows, agg_out.at[c, pl.ds(r0, RCHUNK)])
            pltpu.sync_copy(cnt_sh.at[pl.ds(r0, RCHUNK)], cnt_stage)
            c0 = pl.multiple_of(c * N_NODES + r0, 8)
            pltpu.sync_copy(cnt_stage, cnt_out.at[pl.ds(c0, RCHUNK)])


_sc_aggregate = functools.partial(
    pl.kernel,
    out_type=[
        jax.ShapeDtypeStruct((NC, N_NODES, F_IN), jnp.float32),
        jax.ShapeDtypeStruct((NC * N_NODES,), jnp.float32),
    ],
    mesh=plsc.VectorSubcoreMesh(core_axis_name="c", subcore_axis_name="s"),
    scratch_types=[
        pltpu.VMEM_SHARED((N_NODES, F_IN), jnp.float32),
        pltpu.VMEM_SHARED((N_NODES,), jnp.float32),
        pltpu.VMEM((ECHUNK,), jnp.int32),
        pltpu.VMEM((ECHUNK,), jnp.int32),
        pltpu.VMEM((ECHUNK,), jnp.int32),
        pltpu.VMEM((ECHUNK,), jnp.int32),
        pltpu.VMEM((ECHUNK,), jnp.int32),
        pltpu.VMEM((ECHUNK,), jnp.int32),
        pltpu.VMEM((ECHUNK,), jnp.int32),
        pltpu.VMEM((ECHUNK,), jnp.int32),
        pltpu.VMEM((ECHUNK, F_IN), jnp.float32),
        pltpu.VMEM((ECHUNK, F_IN), jnp.float32),
        pltpu.VMEM((ECHUNK,), jnp.float32),
        pltpu.VMEM((RCHUNK,), jnp.float32),
        pltpu.SemaphoreType.DMA,
        pltpu.SemaphoreType.DMA,
        pltpu.SemaphoreType.DMA,
        pltpu.SemaphoreType.DMA,
        pltpu.SemaphoreType.DMA,
        pltpu.SemaphoreType.DMA,
        pltpu.SemaphoreType.DMA,
        pltpu.SemaphoreType.DMA,
    ],
    compiler_params=pltpu.CompilerParams(use_tc_tiling_on_sc=False),
)(_sc_body)


def _tc_body(agg_ref, cnt_ref, x_ref, batch_ref, wl_ref, bl_ref, wr_ref,
             wlin_ref, blin_ref, out_ref, acc_ref):
    i = pl.program_id(0)

    @pl.when(i == 0)
    def _():
        acc_ref[...] = jnp.zeros_like(acc_ref)

    agg = agg_ref[0] + agg_ref[1]
    cnt = cnt_ref[0] + cnt_ref[1]
    mean = agg / jnp.maximum(cnt, 1.0)
    ids = batch_ref[0]

    node = i * BLK + lax.broadcasted_iota(jnp.int32, (BLK, 1), 0)
    valid = node < N_NODES

    gid = lax.broadcasted_iota(jnp.int32, (BLK, N_GRAPHS), 1)
    onehot = jnp.where((ids == gid) & valid, 1.0, 0.0)

    feat = jnp.concatenate(
        [mean, x_ref[...], jnp.ones((BLK, 1), jnp.float32),
         jnp.zeros((BLK, FW - 2 * F_IN - 1), jnp.float32)], axis=1)
    feat = jnp.where(valid, feat, 0.0)

    acc_ref[...] += lax.dot_general(
        onehot, feat, (((0,), (0,)), ((), ())),
        preferred_element_type=jnp.float32)

    @pl.when(i == NB - 1)
    def _():
        acc = acc_ref[...]
        pmean = acc[:, 0:F_IN]
        px = acc[:, F_IN:2 * F_IN]
        gcnt = acc[:, 2 * F_IN:2 * F_IN + 1]
        gsum = (
            lax.dot_general(pmean, wl_ref[...], (((1,), (1,)), ((), ())),
                            preferred_element_type=jnp.float32)
            + lax.dot_general(px, wr_ref[...], (((1,), (1,)), ((), ())),
                              preferred_element_type=jnp.float32)
            + gcnt * bl_ref[...])
        gmean = gsum / jnp.maximum(gcnt, 1.0)
        out_ref[...] = (
            lax.dot_general(gmean, wlin_ref[...], (((1,), (1,)), ((), ())),
                            preferred_element_type=jnp.float32)
            + blin_ref[...])


def _tc_finish(agg2, cnt2, x, batch3, w_l, b_l, w_r, w_lin, b_lin):
    return pl.pallas_call(
        _tc_body,
        grid=(NB,),
        in_specs=[
            pl.BlockSpec((NC, BLK, F_IN), lambda i: (0, i, 0)),
            pl.BlockSpec((NC, BLK, 1), lambda i: (0, i, 0)),
            pl.BlockSpec((BLK, F_IN), lambda i: (i, 0)),
            pl.BlockSpec((1, BLK, 1), lambda i: (i, 0, 0)),
            pl.BlockSpec((32, 16), lambda i: (0, 0)),
            pl.BlockSpec((1, 32), lambda i: (0, 0)),
            pl.BlockSpec((32, 16), lambda i: (0, 0)),
            pl.BlockSpec((64, 32), lambda i: (0, 0)),
            pl.BlockSpec((1, 64), lambda i: (0, 0)),
        ],
        out_specs=pl.BlockSpec((N_GRAPHS, 64), lambda i: (0, 0)),
        out_shape=jax.ShapeDtypeStruct((N_GRAPHS, 64), jnp.float32),
        scratch_shapes=[pltpu.VMEM((N_GRAPHS, FW), jnp.float32)],
    )(agg2, cnt2, x, batch3, w_l, b_l, w_r, w_lin, b_lin)


def kernel(x, edge_index, batch, W_l, b_l, W_r, W_lin, b_lin):
    src = edge_index[0].astype(jnp.int32)
    dst = edge_index[1].astype(jnp.int32)
    agg2, cnt2 = _sc_aggregate(src, dst, x)
    cnt2 = cnt2.reshape(NC, N_NODES, 1)
    batch_pad = jnp.concatenate(
        [batch.astype(jnp.int32),
         jnp.full((NPAD - N_NODES,), N_GRAPHS, jnp.int32)]).reshape(NB, BLK, 1)
    return _tc_finish(agg2, cnt2, x, batch_pad,
                      W_l, b_l.reshape(1, 32), W_r, W_lin, b_lin.reshape(1, 64))
